# baseline (speedup 1.0000x reference)
import functools

import jax
import jax.numpy as jnp
from jax.experimental import pallas as pl


_B, _C = 16384, 1000
_ROWS = 512


def _body(x_ref, t_ref, out_ref):
    i = pl.program_id(0)
    x = x_ref[...]
    t = t_ref[...]
    masked = jnp.where(t > 0.0, x, -jnp.inf)
    m = jnp.max(masked, axis=1, keepdims=True)
    hp = jnp.max(t, axis=1, keepdims=True) > 0.0
    sig = jnp.clip(jax.nn.sigmoid(m), 1e-6, 1.0 - 1e-6)
    li = jnp.where(hp, -jnp.log(sig), 0.0)
    part = jnp.sum(li, axis=(0, 1), keepdims=True)

    @pl.when(i == 0)
    def _():
        out_ref[...] = jnp.zeros_like(out_ref)

    out_ref[...] += part


@jax.jit
def kernel(input, target):
    B, C = input.shape
    grid = (B // _ROWS,)
    out = pl.pallas_call(
        _body,
        grid=grid,
        in_specs=[
            pl.BlockSpec((_ROWS, C), lambda i: (i, 0)),
            pl.BlockSpec((_ROWS, C), lambda i: (i, 0)),
        ],
        out_specs=pl.BlockSpec((1, 1), lambda i: (0, 0)),
        out_shape=jax.ShapeDtypeStruct((1, 1), jnp.float32),
    )(input, target)
    return out[0, 0] / B
